# baseline (speedup 1.0000x reference)
import functools

import jax
import jax.numpy as jnp
from jax import lax
from jax.experimental import pallas as pl
from jax.experimental.pallas import tpu as pltpu
from jax.experimental.pallas import tpu_sc as plsc

D_MODEL = 128
N_SITES = 512
ROW_LEN = 3 * N_SITES
L = 16
CHUNK = 128


def _make_sc_kernel(batch: int):
    info = plsc.get_sparse_core_info()
    nc, ns = info.num_cores, info.num_subcores
    nw = nc * ns
    assert batch % nw == 0
    rows_per_w = batch // nw
    n_chunks = N_SITES // CHUNK

    mesh = plsc.VectorSubcoreMesh(core_axis_name="c", subcore_axis_name="s")

    @functools.partial(
        pl.kernel,
        mesh=mesh,
        out_type=jax.ShapeDtypeStruct((batch * N_SITES, D_MODEL), jnp.float32),
        scratch_types=[
            pltpu.VMEM((ROW_LEN,), jnp.int32),
            pltpu.VMEM((n_chunks, CHUNK), jnp.int32),
            pltpu.VMEM((CHUNK, D_MODEL), jnp.float32),
            pltpu.SemaphoreType.DMA,
        ],
    )
    def body(n_hbm, emb_hbm, out_hbm, nrow_v, tok_v, rows_v, sem):
        wid = lax.axis_index("s") * nc + lax.axis_index("c")

        def row_loop(r, carry):
            row = wid * rows_per_w + r
            pltpu.sync_copy(n_hbm.at[pl.ds(row * ROW_LEN, ROW_LEN)], nrow_v)
            for c in range(n_chunks):
                for j in range(CHUNK // L):
                    o = c * CHUNK + j * L
                    down = nrow_v[pl.ds(o, L)]
                    up = nrow_v[pl.ds(N_SITES + o, L)]
                    sp = nrow_v[pl.ds(2 * N_SITES + o, L)]
                    tok_v[c, pl.ds(j * L, L)] = (
                        up + 2 * down + 4 * ((sp + 1) >> 1)
                    )
            for c in range(n_chunks):
                pltpu.async_copy(emb_hbm.at[tok_v.at[c]], rows_v, sem).wait()
                pltpu.sync_copy(
                    rows_v, out_hbm.at[pl.ds(row * N_SITES + c * CHUNK, CHUNK)]
                )
            return carry

        lax.fori_loop(0, rows_per_w, row_loop, 0)

    return body


def kernel(n_flat, embedding):
    n = jnp.asarray(n_flat)
    if n.ndim == 1:
        n = n[None, :]
    batch = n.shape[0]
    body = _make_sc_kernel(batch)
    out = body(n.reshape(-1), embedding)
    return out.reshape(batch, N_SITES, D_MODEL)
